# baseline (speedup 1.0000x reference)
import functools

import jax
import jax.numpy as jnp
from jax import lax
from jax.experimental import pallas as pl
from jax.experimental.pallas import tpu as pltpu
from jax.experimental.pallas import tpu_sc as plsc

GAMMA = 12.0
BATCH = 4096
DIM = 128
LANES = 16
NUM_CORES = 2
NUM_SUBCORES = 16
NW = NUM_CORES * NUM_SUBCORES
BPW = BATCH // NW
CHUNKS = DIM // LANES
STRIDE = LANES + 1
NB = 4
BROWS = BPW // NB


def _transe_body(entity_hbm, relation_hbm, idx_hbm,
                 out_hbm,
                 idx_v, h_rows, r_rows, t_rows,
                 partials, out_v, sem_i, sem_a, sem_b):
    wid = lax.axis_index("s") * NUM_CORES + lax.axis_index("c")
    base = wid * BPW

    pltpu.async_copy(idx_hbm.at[pl.ds(wid * 3 * BPW, 3 * BPW)], idx_v,
                     sem_i).wait()

    def block_copies(b, sem):
        s = pl.ds(b * BROWS, BROWS)
        return [
            pltpu.make_async_copy(
                entity_hbm.at[idx_v.at[pl.ds(b * BROWS, BROWS)]],
                h_rows.at[s, :], sem),
            pltpu.make_async_copy(
                relation_hbm.at[idx_v.at[pl.ds(BPW + b * BROWS, BROWS)]],
                r_rows.at[s, :], sem),
            pltpu.make_async_copy(
                entity_hbm.at[idx_v.at[pl.ds(2 * BPW + b * BROWS, BROWS)]],
                t_rows.at[s, :], sem),
        ]

    def fire(b, sem):
        for c in block_copies(b, sem):
            c.start()

    def drain(b, sem):
        for c in block_copies(b, sem):
            c.wait()

    fire(0, sem_a)
    fire(1, sem_b)

    lane = lax.iota(jnp.int32, LANES)

    def block_body(b, carry):
        even = b % 2 == 0

        @pl.when(even)
        def _():
            drain(b, sem_a)

        @pl.when(jnp.logical_not(even))
        def _():
            drain(b, sem_b)

        @pl.when(jnp.logical_and(even, b < NB - 2))
        def _():
            fire(b + 2, sem_a)

        @pl.when(jnp.logical_and(jnp.logical_not(even), b < NB - 2))
        def _():
            fire(b + 2, sem_b)

        def row_body(i, carry2):
            row = b * BROWS + i
            acc0 = jnp.zeros((LANES,), jnp.float32)
            acc1 = jnp.zeros((LANES,), jnp.float32)
            for c in range(0, CHUNKS, 2):
                hh = h_rows[row, pl.ds(c * LANES, LANES)]
                rr = r_rows[row, pl.ds(c * LANES, LANES)]
                tt = t_rows[row, pl.ds(c * LANES, LANES)]
                acc0 = acc0 + jnp.abs(hh + rr - tt)
                hh = h_rows[row, pl.ds((c + 1) * LANES, LANES)]
                rr = r_rows[row, pl.ds((c + 1) * LANES, LANES)]
                tt = t_rows[row, pl.ds((c + 1) * LANES, LANES)]
                acc1 = acc1 + jnp.abs(hh + rr - tt)
            partials[pl.ds(row * STRIDE, LANES)] = acc0 + acc1
            return carry2

        lax.fori_loop(0, BROWS, row_body, 0, unroll=2)

        for g in range(BROWS // LANES):
            rows = (lane + b * BROWS + g * LANES) * STRIDE
            tot = jnp.zeros((LANES,), jnp.float32)
            for c in range(LANES):
                tot = tot + plsc.load_gather(partials, [rows + c])
            out_v[pl.ds(b * BROWS + g * LANES, LANES)] = GAMMA - tot
        return carry

    lax.fori_loop(0, NB, block_body, 0)

    pltpu.sync_copy(out_v, out_hbm.at[pl.ds(base, BPW)])


_transe_sc = functools.partial(
    pl.kernel,
    mesh=plsc.VectorSubcoreMesh(core_axis_name="c", subcore_axis_name="s"),
    out_type=jax.ShapeDtypeStruct((BATCH,), jnp.float32),
    compiler_params=pltpu.CompilerParams(needs_layout_passes=False),
    scratch_types=[
        pltpu.VMEM((3 * BPW,), jnp.int32),
        pltpu.VMEM((BPW, DIM), jnp.float32),
        pltpu.VMEM((BPW, DIM), jnp.float32),
        pltpu.VMEM((BPW, DIM), jnp.float32),
        pltpu.VMEM((BPW * STRIDE,), jnp.float32),
        pltpu.VMEM((BPW,), jnp.float32),
        pltpu.SemaphoreType.DMA,
        pltpu.SemaphoreType.DMA,
        pltpu.SemaphoreType.DMA,
    ],
)(_transe_body)


@jax.jit
def kernel(sample, entity_embedding, relation_embedding):
    idx_packed = jnp.concatenate(
        [sample[:, 0].reshape(NW, BPW), sample[:, 1].reshape(NW, BPW),
         sample[:, 2].reshape(NW, BPW)], axis=1).reshape(3 * BATCH)
    score = _transe_sc(entity_embedding, relation_embedding, idx_packed)
    return score.reshape(BATCH, 1)


# speedup vs baseline: 1.0141x; 1.0141x over previous
import functools

import jax
import jax.numpy as jnp
from jax import lax
from jax.experimental import pallas as pl
from jax.experimental.pallas import tpu as pltpu
from jax.experimental.pallas import tpu_sc as plsc

GAMMA = 12.0
BATCH = 4096
DIM = 128
LANES = 16
NUM_CORES = 2
NUM_SUBCORES = 16
NW = NUM_CORES * NUM_SUBCORES
BPW = BATCH // NW
CHUNKS = DIM // LANES
STRIDE = LANES + 1
NB = 8
BROWS = BPW // NB


def _transe_body(entity_hbm, relation_hbm, idx_hbm,
                 out_hbm,
                 idx_v, h_rows, r_rows, t_rows,
                 partials, out_v, sem_i, sem_a, sem_b):
    wid = lax.axis_index("s") * NUM_CORES + lax.axis_index("c")
    base = wid * BPW

    pltpu.async_copy(idx_hbm.at[pl.ds(wid * 3 * BPW, 3 * BPW)], idx_v,
                     sem_i).wait()

    def block_copies(b, sem):
        s = pl.ds(b * BROWS, BROWS)
        return [
            pltpu.make_async_copy(
                entity_hbm.at[idx_v.at[pl.ds(b * BROWS, BROWS)]],
                h_rows.at[s, :], sem),
            pltpu.make_async_copy(
                relation_hbm.at[idx_v.at[pl.ds(BPW + b * BROWS, BROWS)]],
                r_rows.at[s, :], sem),
            pltpu.make_async_copy(
                entity_hbm.at[idx_v.at[pl.ds(2 * BPW + b * BROWS, BROWS)]],
                t_rows.at[s, :], sem),
        ]

    def fire(b, sem):
        for c in block_copies(b, sem):
            c.start()

    def drain(b, sem):
        for c in block_copies(b, sem):
            c.wait()

    fire(0, sem_a)
    fire(1, sem_b)

    lane = lax.iota(jnp.int32, LANES)

    def block_body(b, carry):
        even = b % 2 == 0

        @pl.when(even)
        def _():
            drain(b, sem_a)

        @pl.when(jnp.logical_not(even))
        def _():
            drain(b, sem_b)

        @pl.when(jnp.logical_and(even, b < NB - 2))
        def _():
            fire(b + 2, sem_a)

        @pl.when(jnp.logical_and(jnp.logical_not(even), b < NB - 2))
        def _():
            fire(b + 2, sem_b)

        def row_body(i, carry2):
            row = b * BROWS + i
            acc0 = jnp.zeros((LANES,), jnp.float32)
            acc1 = jnp.zeros((LANES,), jnp.float32)
            for c in range(0, CHUNKS, 2):
                hh = h_rows[row, pl.ds(c * LANES, LANES)]
                rr = r_rows[row, pl.ds(c * LANES, LANES)]
                tt = t_rows[row, pl.ds(c * LANES, LANES)]
                acc0 = acc0 + jnp.abs(hh + rr - tt)
                hh = h_rows[row, pl.ds((c + 1) * LANES, LANES)]
                rr = r_rows[row, pl.ds((c + 1) * LANES, LANES)]
                tt = t_rows[row, pl.ds((c + 1) * LANES, LANES)]
                acc1 = acc1 + jnp.abs(hh + rr - tt)
            partials[pl.ds(row * STRIDE, LANES)] = acc0 + acc1
            return carry2

        lax.fori_loop(0, BROWS, row_body, 0, unroll=2)

        for g in range(BROWS // LANES):
            rows = (lane + b * BROWS + g * LANES) * STRIDE
            tot = jnp.zeros((LANES,), jnp.float32)
            for c in range(LANES):
                tot = tot + plsc.load_gather(partials, [rows + c])
            out_v[pl.ds(b * BROWS + g * LANES, LANES)] = GAMMA - tot
        return carry

    lax.fori_loop(0, NB, block_body, 0)

    pltpu.sync_copy(out_v, out_hbm.at[pl.ds(base, BPW)])


_transe_sc = functools.partial(
    pl.kernel,
    mesh=plsc.VectorSubcoreMesh(core_axis_name="c", subcore_axis_name="s"),
    out_type=jax.ShapeDtypeStruct((BATCH,), jnp.float32),
    compiler_params=pltpu.CompilerParams(needs_layout_passes=False),
    scratch_types=[
        pltpu.VMEM((3 * BPW,), jnp.int32),
        pltpu.VMEM((BPW, DIM), jnp.float32),
        pltpu.VMEM((BPW, DIM), jnp.float32),
        pltpu.VMEM((BPW, DIM), jnp.float32),
        pltpu.VMEM((BPW * STRIDE,), jnp.float32),
        pltpu.VMEM((BPW,), jnp.float32),
        pltpu.SemaphoreType.DMA,
        pltpu.SemaphoreType.DMA,
        pltpu.SemaphoreType.DMA,
    ],
)(_transe_body)


@jax.jit
def kernel(sample, entity_embedding, relation_embedding):
    idx_packed = jnp.concatenate(
        [sample[:, 0].reshape(NW, BPW), sample[:, 1].reshape(NW, BPW),
         sample[:, 2].reshape(NW, BPW)], axis=1).reshape(3 * BATCH)
    score = _transe_sc(entity_embedding, relation_embedding, idx_packed)
    return score.reshape(BATCH, 1)


# speedup vs baseline: 1.0328x; 1.0185x over previous
import functools

import jax
import jax.numpy as jnp
from jax import lax
from jax.experimental import pallas as pl
from jax.experimental.pallas import tpu as pltpu
from jax.experimental.pallas import tpu_sc as plsc

GAMMA = 12.0
BATCH = 4096
DIM = 128
LANES = 16
NUM_CORES = 2
NUM_SUBCORES = 16
NW = NUM_CORES * NUM_SUBCORES
BPW = BATCH // NW
CHUNKS = DIM // LANES
STRIDE = LANES + 1
NB = 8
BROWS = BPW // NB


def _transe_body(entity_hbm, relation_hbm, idx_hbm,
                 out_hbm,
                 idx_v, h_rows, r_rows, t_rows,
                 partials, out_v, sem_i, sem_a, sem_b, sem_c):
    wid = lax.axis_index("s") * NUM_CORES + lax.axis_index("c")
    base = wid * BPW

    pltpu.async_copy(idx_hbm.at[pl.ds(wid * 3 * BPW, 3 * BPW)], idx_v,
                     sem_i).wait()

    def block_copies(b, sem):
        s = pl.ds(b * BROWS, BROWS)
        return [
            pltpu.make_async_copy(
                entity_hbm.at[idx_v.at[pl.ds(b * BROWS, BROWS)]],
                h_rows.at[s, :], sem),
            pltpu.make_async_copy(
                relation_hbm.at[idx_v.at[pl.ds(BPW + b * BROWS, BROWS)]],
                r_rows.at[s, :], sem),
            pltpu.make_async_copy(
                entity_hbm.at[idx_v.at[pl.ds(2 * BPW + b * BROWS, BROWS)]],
                t_rows.at[s, :], sem),
        ]

    def fire(b, sem):
        for c in block_copies(b, sem):
            c.start()

    def drain(b, sem):
        for c in block_copies(b, sem):
            c.wait()

    fire(0, sem_a)
    fire(1, sem_b)
    fire(2, sem_c)

    lane = lax.iota(jnp.int32, LANES)

    def block_body(b, carry):
        par = b % 3

        @pl.when(par == 0)
        def _():
            drain(b, sem_a)

        @pl.when(par == 1)
        def _():
            drain(b, sem_b)

        @pl.when(par == 2)
        def _():
            drain(b, sem_c)

        @pl.when(jnp.logical_and(par == 0, b < NB - 3))
        def _():
            fire(b + 3, sem_a)

        @pl.when(jnp.logical_and(par == 1, b < NB - 3))
        def _():
            fire(b + 3, sem_b)

        @pl.when(jnp.logical_and(par == 2, b < NB - 3))
        def _():
            fire(b + 3, sem_c)

        def row_body(i, carry2):
            row = b * BROWS + i
            acc0 = jnp.zeros((LANES,), jnp.float32)
            acc1 = jnp.zeros((LANES,), jnp.float32)
            for c in range(0, CHUNKS, 2):
                hh = h_rows[row, pl.ds(c * LANES, LANES)]
                rr = r_rows[row, pl.ds(c * LANES, LANES)]
                tt = t_rows[row, pl.ds(c * LANES, LANES)]
                acc0 = acc0 + jnp.abs(hh + rr - tt)
                hh = h_rows[row, pl.ds((c + 1) * LANES, LANES)]
                rr = r_rows[row, pl.ds((c + 1) * LANES, LANES)]
                tt = t_rows[row, pl.ds((c + 1) * LANES, LANES)]
                acc1 = acc1 + jnp.abs(hh + rr - tt)
            partials[pl.ds(row * STRIDE, LANES)] = acc0 + acc1
            return carry2

        lax.fori_loop(0, BROWS, row_body, 0, unroll=2)

        for g in range(BROWS // LANES):
            rows = (lane + b * BROWS + g * LANES) * STRIDE
            tot = jnp.zeros((LANES,), jnp.float32)
            for c in range(LANES):
                tot = tot + plsc.load_gather(partials, [rows + c])
            out_v[pl.ds(b * BROWS + g * LANES, LANES)] = GAMMA - tot
        return carry

    lax.fori_loop(0, NB, block_body, 0)

    pltpu.sync_copy(out_v, out_hbm.at[pl.ds(base, BPW)])


_transe_sc = functools.partial(
    pl.kernel,
    mesh=plsc.VectorSubcoreMesh(core_axis_name="c", subcore_axis_name="s"),
    out_type=jax.ShapeDtypeStruct((BATCH,), jnp.float32),
    compiler_params=pltpu.CompilerParams(needs_layout_passes=False),
    scratch_types=[
        pltpu.VMEM((3 * BPW,), jnp.int32),
        pltpu.VMEM((BPW, DIM), jnp.float32),
        pltpu.VMEM((BPW, DIM), jnp.float32),
        pltpu.VMEM((BPW, DIM), jnp.float32),
        pltpu.VMEM((BPW * STRIDE,), jnp.float32),
        pltpu.VMEM((BPW,), jnp.float32),
        pltpu.SemaphoreType.DMA,
        pltpu.SemaphoreType.DMA,
        pltpu.SemaphoreType.DMA,
        pltpu.SemaphoreType.DMA,
    ],
)(_transe_body)


@jax.jit
def kernel(sample, entity_embedding, relation_embedding):
    idx_packed = jnp.concatenate(
        [sample[:, 0].reshape(NW, BPW), sample[:, 1].reshape(NW, BPW),
         sample[:, 2].reshape(NW, BPW)], axis=1).reshape(3 * BATCH)
    score = _transe_sc(entity_embedding, relation_embedding, idx_packed)
    return score.reshape(BATCH, 1)
